# baseline (speedup 1.0000x reference)
import functools

import jax
import jax.numpy as jnp
from jax import lax
from jax.experimental import pallas as pl
from jax.experimental.pallas import tpu as pltpu
from jax.experimental.pallas import tpu_sc as plsc

NUM_SCENES = 1000000
LATENT = 64
BATCH = 16384
LANES = 16
NUM_CORES = 2
NUM_SUBCORES = 16
NUM_WORKERS = NUM_CORES * NUM_SUBCORES
BPW = BATCH // NUM_WORKERS
PACK = 128
VECS_PER_ROW = LATENT // LANES
CHUNK = 16

_GATHER_DNUMS = lax.GatherDimensionNumbers(
    offset_dims=(), collapsed_slice_dims=(0,), start_index_map=(0,)
)


def _permute(v, idx):
    return lax.gather(
        v,
        idx[:, None],
        _GATHER_DNUMS,
        (1,),
        mode=lax.GatherScatterMode.PROMISE_IN_BOUNDS,
    )


def _rsqrt(x):
    i = lax.bitcast_convert_type(x, jnp.int32)
    i = jnp.int32(0x5F3759DF) - lax.shift_right_arithmetic(i, 1)
    y = lax.bitcast_convert_type(i, jnp.float32)
    for _ in range(3):
        y = y * (1.5 - 0.5 * x * y * y)
    return y


@functools.partial(
    pl.kernel,
    out_type=jax.ShapeDtypeStruct((BATCH, LATENT), jnp.float32),
    mesh=plsc.VectorSubcoreMesh(core_axis_name="c", subcore_axis_name="s"),
    scratch_types=[
        pltpu.VMEM((BPW,), jnp.int32),
        pltpu.VMEM((BPW // 2, PACK), jnp.float32),
        pltpu.VMEM((BPW, LATENT), jnp.float32),
        pltpu.SemaphoreType.DMA,
    ],
)
def _gather_maxnorm(idx_hbm, table_hbm, out_hbm, idx_v, rows_v, out_v, sem):
    wid = lax.axis_index("s") * NUM_CORES + lax.axis_index("c")
    base = wid * BPW
    pltpu.sync_copy(idx_hbm.at[pl.ds(base, BPW)], idx_v)

    lanes = lax.iota(jnp.int32, LANES)
    perms = [lanes ^ sh for sh in (8, 4, 2, 1)]
    half = BPW // 2

    def half_pass(h):
        h0 = h * half

        def gather_chunk(c, carry):
            r0 = c * CHUNK
            ivec = idx_v[pl.ds(h0 + r0, CHUNK)]
            cps = []
            for j in range(CHUNK):
                i = ivec[j]
                cps.append(
                    pltpu.async_copy(
                        table_hbm.at[
                            pl.ds(lax.shift_right_logical(i, 1), 1), :
                        ],
                        rows_v.at[pl.ds(r0 + j, 1), :],
                        sem,
                    )
                )
            for cp in cps:
                cp.wait()
            return carry

        lax.fori_loop(0, half // CHUNK, gather_chunk, 0)

        def row_chunk(c, carry):
            r0 = c * CHUNK
            ivec = idx_v[pl.ds(h0 + r0, CHUNK)]
            for j in range(CHUNK):
                r = r0 + j
                off = (ivec[j] & 1) * LATENT
                vecs = [
                    rows_v[r, pl.ds(off + q * LANES, LANES)]
                    for q in range(VECS_PER_ROW)
                ]
                acc = vecs[0] * vecs[0]
                for v in vecs[1:]:
                    acc = acc + v * v
                for p in perms:
                    acc = acc + _permute(acc, p)
                scale = jnp.minimum(1.0, _rsqrt(acc))
                for q in range(VECS_PER_ROW):
                    out_v[h0 + r, pl.ds(q * LANES, LANES)] = vecs[q] * scale
            return carry

        lax.fori_loop(0, half // CHUNK, row_chunk, 0)

    half_pass(0)
    half_pass(1)
    pltpu.sync_copy(out_v, out_hbm.at[pl.ds(base, BPW), :])


def kernel(idxs, table):
    return _gather_maxnorm(
        idxs.astype(jnp.int32), jnp.reshape(table, (NUM_SCENES // 2, 2 * LATENT))
    )


# speedup vs baseline: 1.6471x; 1.6471x over previous
import functools

import jax
import jax.numpy as jnp
from jax import lax
from jax.experimental import pallas as pl
from jax.experimental.pallas import tpu as pltpu
from jax.experimental.pallas import tpu_sc as plsc

NUM_SCENES = 1000000
LATENT = 64
BATCH = 16384
LANES = 16
NUM_CORES = 2
NUM_SUBCORES = 16
NUM_WORKERS = NUM_CORES * NUM_SUBCORES
BPW = BATCH // NUM_WORKERS
VECS_PER_ROW = LATENT // LANES
CHUNK = 16

_GATHER_DNUMS = lax.GatherDimensionNumbers(
    offset_dims=(), collapsed_slice_dims=(0,), start_index_map=(0,)
)


def _permute(v, idx):
    return lax.gather(
        v,
        idx[:, None],
        _GATHER_DNUMS,
        (1,),
        mode=lax.GatherScatterMode.PROMISE_IN_BOUNDS,
    )


def _rsqrt(x):
    i = lax.bitcast_convert_type(x, jnp.int32)
    i = jnp.int32(0x5F3759DF) - lax.shift_right_arithmetic(i, 1)
    y = lax.bitcast_convert_type(i, jnp.float32)
    for _ in range(3):
        y = y * (1.5 - 0.5 * x * y * y)
    return y


@functools.partial(
    pl.kernel,
    out_type=jax.ShapeDtypeStruct((BATCH, LATENT), jnp.float32),
    mesh=plsc.VectorSubcoreMesh(core_axis_name="c", subcore_axis_name="s"),
    scratch_types=[
        pltpu.VMEM((BPW,), jnp.int32),
        pltpu.VMEM((BPW, LATENT), jnp.float32),
        pltpu.SemaphoreType.DMA,
    ],
)
def _gather_maxnorm(idx_hbm, table_hbm, out_hbm, idx_v, rows_v, sem):
    wid = lax.axis_index("s") * NUM_CORES + lax.axis_index("c")
    base = wid * BPW
    pltpu.sync_copy(idx_hbm.at[pl.ds(base, BPW)], idx_v)

    def gather_chunk(c, carry):
        r0 = c * CHUNK
        ivec = idx_v[pl.ds(r0, CHUNK)]
        cps = []
        for j in range(CHUNK):
            i = ivec[j]
            cps.append(
                pltpu.async_copy(
                    table_hbm.at[pl.ds(i, 1), :],
                    rows_v.at[pl.ds(r0 + j, 1), :],
                    sem,
                )
            )
        for cp in cps:
            cp.wait()
        return carry

    lax.fori_loop(0, BPW // CHUNK, gather_chunk, 0)

    lanes = lax.iota(jnp.int32, LANES)
    perms = [lanes ^ sh for sh in (8, 4, 2, 1)]

    def row_fn(r, carry):
        vecs = [rows_v[r, pl.ds(j * LANES, LANES)] for j in range(VECS_PER_ROW)]
        acc = vecs[0] * vecs[0]
        for v in vecs[1:]:
            acc = acc + v * v
        for p in perms:
            acc = acc + _permute(acc, p)
        scale = jnp.minimum(1.0, _rsqrt(acc))
        for j in range(VECS_PER_ROW):
            rows_v[r, pl.ds(j * LANES, LANES)] = vecs[j] * scale
        return carry

    lax.fori_loop(0, BPW, row_fn, 0)
    pltpu.sync_copy(rows_v, out_hbm.at[pl.ds(base, BPW), :])


def kernel(idxs, table):
    return _gather_maxnorm(idxs.astype(jnp.int32), table)
